# baseline (speedup 1.0000x reference)
import functools

import jax
import jax.numpy as jnp
from jax import lax
from jax.experimental import pallas as pl
from jax.experimental.pallas import tpu as pltpu
from jax.experimental.pallas import tpu_sc as plsc

NCLS = 1000
NROWS = 4096 * 26
NW = 32
ROWS_PER_W = NROWS // NW
CHUNK_ROWS = 32
CHUNK = CHUNK_ROWS * NCLS
NCHUNK = ROWS_PER_W // CHUNK_ROWS
NBUF = 2
LANES = 16

_mesh = plsc.VectorSubcoreMesh(core_axis_name="c", subcore_axis_name="s")


@functools.partial(
    pl.kernel,
    out_type=jax.ShapeDtypeStruct((NROWS * NCLS,), jnp.float32),
    mesh=_mesh,
    scratch_types=[
        pltpu.VMEM((ROWS_PER_W,), jnp.int32),
        pltpu.VMEM((NBUF * CHUNK,), jnp.float32),
        pltpu.VMEM((NBUF * CHUNK_ROWS,), jnp.int32),
        pltpu.SemaphoreType.DMA,
        pltpu.SemaphoreType.DMA,
    ],
    compiler_params=pltpu.CompilerParams(needs_layout_passes=False),
)
def _sc_onehot(x_hbm, out_hbm, idx_v, buf, offbuf, sem0, sem1):
    wid = lax.axis_index("s") * 2 + lax.axis_index("c")
    row0 = wid * ROWS_PER_W
    out0 = row0 * NCLS
    pltpu.sync_copy(x_hbm.at[pl.ds(row0, ROWS_PER_W)], idx_v)

    zeros = jnp.zeros((LANES,), jnp.float32)
    ones = jnp.ones((LANES,), jnp.float32)

    def zero_body(i, carry):
        buf[pl.ds(i * LANES, LANES)] = zeros
        return carry

    lax.fori_loop(0, NBUF * CHUNK // LANES, zero_body, 0)

    sems = (sem0, sem1)

    def outer(g, carry):
        for b in range(NBUF):
            c = g * NBUF + b

            @pl.when(g >= 1)
            def _wait_and_clear():
                pltpu.make_async_copy(
                    buf.at[pl.ds(b * CHUNK, CHUNK)],
                    out_hbm.at[pl.ds(0, CHUNK)],
                    sems[b],
                ).wait()
                for j in range(CHUNK_ROWS // LANES):
                    old = offbuf[pl.ds(b * CHUNK_ROWS + j * LANES, LANES)]
                    plsc.store_scatter(buf, [old], zeros)

            for j in range(CHUNK_ROWS // LANES):
                ids = idx_v[pl.ds(c * CHUNK_ROWS + j * LANES, LANES)]
                local = (
                    (lax.iota(jnp.int32, LANES) + (j * LANES)) * NCLS
                    + ids
                    + b * CHUNK
                )
                plsc.store_scatter(buf, [local], ones)
                offbuf[pl.ds(b * CHUNK_ROWS + j * LANES, LANES)] = local

            pltpu.async_copy(
                buf.at[pl.ds(b * CHUNK, CHUNK)],
                out_hbm.at[pl.ds(out0 + c * CHUNK, CHUNK)],
                sems[b],
            )
        return carry

    lax.fori_loop(0, NCHUNK // NBUF, outer, 0)

    for b in range(NBUF):
        pltpu.make_async_copy(
            buf.at[pl.ds(b * CHUNK, CHUNK)],
            out_hbm.at[pl.ds(0, CHUNK)],
            sems[b],
        ).wait()


def kernel(x):
    xf = x.reshape(-1).astype(jnp.int32)
    out = _sc_onehot(xf)
    return out.reshape(tuple(x.shape) + (NCLS,))
